# baseline (speedup 1.0000x reference)
import jax
import jax.numpy as jnp
from jax.experimental import pallas as pl
from jax.experimental.pallas import tpu as pltpu

_SIGMOID_LAMBDA = 28.0
_SIGMOID_PHI = 5.0
_CB = 2


def _mac_body(xt_ref, w_ref, g_ref, out_ref):
    xt = xt_ref[...]
    s = jnp.sum(xt, axis=0, keepdims=True)
    rs = jnp.where(s > 0.0, 1.0 / s, 0.0)
    cb, n, k = w_ref.shape
    b = xt.shape[1]
    wl = w_ref[...].reshape(cb * n, k)
    yt = jnp.dot(wl, xt, preferred_element_type=jnp.float32) * rs
    t = jnp.log(1.0 / (1.0 + jnp.exp(-_SIGMOID_LAMBDA * yt + _SIGMOID_PHI)))
    t = (t + g_ref[...]).reshape(cb, n, b)
    m = jnp.max(t, axis=1, keepdims=True)
    iota = jax.lax.broadcasted_iota(jnp.int32, t.shape, 1)
    first = jnp.min(jnp.where(t == m, iota, n), axis=1, keepdims=True)
    out_ref[...] = (iota == first).astype(jnp.float32).reshape(cb * n, b)


def kernel(x, weights):
    b = x.shape[0]
    num_cms, k, n = weights.shape
    xt = x.reshape(b, k).T
    wt = weights.transpose(0, 2, 1)
    g = jax.random.gumbel(jax.random.key(123), (b, num_cms, n), jnp.float32)
    gt = g.transpose(1, 2, 0).reshape(num_cms * n, b)
    out_t = pl.pallas_call(
        _mac_body,
        grid=(num_cms // _CB,),
        in_specs=[
            pl.BlockSpec((k, b), lambda i: (0, 0)),
            pl.BlockSpec((_CB, n, k), lambda i: (i, 0, 0)),
            pl.BlockSpec((_CB * n, b), lambda i: (i, 0)),
        ],
        out_specs=pl.BlockSpec((_CB * n, b), lambda i: (i, 0)),
        out_shape=jax.ShapeDtypeStruct((num_cms * n, b), jnp.float32),
        compiler_params=pltpu.CompilerParams(
            dimension_semantics=("arbitrary",),
            vmem_limit_bytes=100 * 1024 * 1024,
        ),
    )(xt, wt, gt)
    return out_t.reshape(num_cms, n, b).transpose(2, 0, 1)


# speedup vs baseline: 1.3286x; 1.3286x over previous
import jax
import jax.numpy as jnp
from jax.experimental import pallas as pl
from jax.experimental.pallas import tpu as pltpu

_SIGMOID_LAMBDA = 28.0
_SIGMOID_PHI = 5.0
_CB = 4
_NS = 2


def _half(w_ref, g_ref, out_ref, xt, rs, lo):
    cb, n, k = w_ref.shape
    b = xt.shape[1]
    wl = w_ref[...].reshape(cb * n, k)
    yt = jnp.dot(wl, xt, preferred_element_type=jnp.float32) * rs
    t = jnp.log(1.0 / (1.0 + jnp.exp(-_SIGMOID_LAMBDA * yt + _SIGMOID_PHI)))
    t = (t + g_ref[lo:lo + cb * n, :]).reshape(cb, n, b)
    m = jnp.max(t, axis=1, keepdims=True)
    iota = jax.lax.broadcasted_iota(jnp.int32, t.shape, 1)
    first = jnp.min(jnp.where(t == m, iota, n), axis=1, keepdims=True)
    out_ref[lo:lo + cb * n, :] = (
        (iota == first).astype(jnp.float32).reshape(cb * n, b))


def _mac_body(xt_ref, wa_ref, wb_ref, g_ref, out_ref):
    xt = xt_ref[...]
    s = jnp.sum(xt, axis=0, keepdims=True)
    rs = jnp.where(s > 0.0, 1.0 / s, 0.0)
    n = wa_ref.shape[1]
    _half(wa_ref, g_ref, out_ref, xt, rs, 0)
    _half(wb_ref, g_ref, out_ref, xt, rs, _CB * n)


def kernel(x, weights):
    b = x.shape[0]
    num_cms, k, n = weights.shape
    xt = x.reshape(b, k).T
    wt = weights.transpose(0, 2, 1)
    g = jax.random.gumbel(jax.random.key(123), (b, num_cms, n), jnp.float32)
    gt = g.transpose(1, 2, 0).reshape(num_cms * n, b)
    step = _CB * _NS
    out_t = pl.pallas_call(
        _mac_body,
        grid=(num_cms // step,),
        in_specs=[
            pl.BlockSpec((k, b), lambda i: (0, 0)),
            pl.BlockSpec((_CB, n, k), lambda i: (_NS * i, 0, 0)),
            pl.BlockSpec((_CB, n, k), lambda i: (_NS * i + 1, 0, 0)),
            pl.BlockSpec((step * n, b), lambda i: (i, 0)),
        ],
        out_specs=pl.BlockSpec((step * n, b), lambda i: (i, 0)),
        out_shape=jax.ShapeDtypeStruct((num_cms * n, b), jnp.float32),
        compiler_params=pltpu.CompilerParams(
            dimension_semantics=("arbitrary",),
            vmem_limit_bytes=100 * 1024 * 1024,
        ),
    )(xt, wt, wt, gt)
    return out_t.reshape(num_cms, n, b).transpose(2, 0, 1)
